# baseline (device time: 62420 ns/iter reference)
import jax
import jax.numpy as jnp
from jax import lax
from jax.experimental import pallas as pl
from jax.experimental.pallas import tpu as pltpu

NZ = 4
T = 256
QB = T // 4
FB = QB // 2
D = 4096
N_FULL = NZ * D
NH = NZ - 1
S = 4
SD = D // S


def kernel(x, W):
    def body(x_ref, w_ref, out_ref, g_ref,
             zs, zr, xds, xdr, yds, ydr, xfs, xfr, yfs, yfr):
        my_x = lax.axis_index("x")
        my_y = lax.axis_index("y")
        my_z = lax.axis_index("z")
        left = (my_z - 1) % NZ
        right = (my_z + 1) % NZ
        r = 2 * my_x + my_y
        r_x = 2 * (1 - my_x) + my_y
        r_y = 2 * my_x + (1 - my_y)

        barrier_sem = pltpu.get_barrier_semaphore()
        for dev in ((my_x, my_y, left), (my_x, my_y, right),
                    (1 - my_x, my_y, my_z), (my_x, 1 - my_y, my_z)):
            pl.semaphore_signal(
                barrier_sem, inc=1,
                device_id=dev, device_id_type=pl.DeviceIdType.MESH,
            )
        pl.semaphore_wait(barrier_sem, 4)

        def desc(src, dst, send_sem, recv_sem, dev):
            return pltpu.make_async_remote_copy(
                src_ref=src, dst_ref=dst,
                send_sem=send_sem, recv_sem=recv_sem,
                device_id=dev, device_id_type=pl.DeviceIdType.MESH,
            )

        x_nbr = (1 - my_x, my_y, my_z)
        y_nbr = (my_x, 1 - my_y, my_z)

        def ring_desc(h, s):
            c = (my_z - h) % NZ
            blk = g_ref.at[c, s, r]
            return desc(blk, blk, zs.at[h, s], zr.at[h, s],
                        (my_x, my_y, right))

        def xdir_desc(h, s):
            c = (my_z - h - 1) % NZ
            blk = g_ref.at[c, s, r]
            return desc(blk, blk, xds.at[h, s], xdr.at[h, s], x_nbr)

        def ydir_desc(h, s):
            c = (my_z - h - 1) % NZ
            blk = g_ref.at[c, s, r]
            return desc(blk, blk, yds.at[h, s], ydr.at[h, s], y_nbr)

        def xfwd_desc(h, s):
            c = (my_z - h - 1) % NZ
            blk = g_ref.at[c, s, r_y, pl.ds(0, FB)]
            return desc(blk, blk, xfs.at[h, s], xfr.at[h, s], x_nbr)

        def yfwd_desc(h, s):
            c = (my_z - h - 1) % NZ
            blk = g_ref.at[c, s, r_x, pl.ds(FB, FB)]
            return desc(blk, blk, yfs.at[h, s], yfr.at[h, s], y_nbr)

        x_bf = x_ref[...].astype(jnp.bfloat16)
        ring = {}
        for s in range(S):
            logits = jnp.dot(x_bf,
                             w_ref[:, s * SD:(s + 1) * SD].astype(
                                 jnp.bfloat16),
                             preferred_element_type=jnp.float32)
            e_s = jnp.exp(logits).astype(jnp.bfloat16)
            for q in range(4):
                g_ref[my_z, s, q] = e_s[q * QB:(q + 1) * QB, :]
            ring[(0, s)] = ring_desc(0, s)
            ring[(0, s)].start()

        xdir, ydir, xfwd, yfwd = {}, {}, {}, {}
        for h in range(NH):
            for s in range(S):
                ring[(h, s)].wait_recv()
                if h + 1 < NH:
                    ring[(h + 1, s)] = ring_desc(h + 1, s)
                    ring[(h + 1, s)].start()
                xdir[(h, s)] = xdir_desc(h, s)
                xdir[(h, s)].start()
                ydir[(h, s)] = ydir_desc(h, s)
                ydir[(h, s)].start()
                if h >= 1:
                    xdir[(h - 1, s)].wait_recv()
                    yfwd[(h - 1, s)] = yfwd_desc(h - 1, s)
                    yfwd[(h - 1, s)].start()
                    ydir[(h - 1, s)].wait_recv()
                    xfwd[(h - 1, s)] = xfwd_desc(h - 1, s)
                    xfwd[(h - 1, s)].start()

        for s in range(S):
            xdir[(NH - 1, s)].wait_recv()
            yfwd[(NH - 1, s)] = yfwd_desc(NH - 1, s)
            yfwd[(NH - 1, s)].start()
            ydir[(NH - 1, s)].wait_recv()
            xfwd[(NH - 1, s)] = xfwd_desc(NH - 1, s)
            xfwd[(NH - 1, s)].start()

        for h in range(NH):
            for s in range(S):
                xfwd[(h, s)].wait_recv()
                yfwd[(h, s)].wait_recv()

        for ds in (ring, xdir, ydir, xfwd, yfwd):
            for d in ds.values():
                d.wait_send()

        accs = []
        for q in range(4):
            aq = jnp.zeros((QB, 1), jnp.float32)
            for c in range(NZ):
                for s in range(S):
                    aq = aq + jnp.sum(
                        g_ref[c, s, q].astype(jnp.float32),
                        axis=1, keepdims=True)
            accs.append(aq)
        inv = 1.0 / jnp.concatenate(accs, axis=0)

        for c in range(NZ):
            for s in range(S):
                for q in range(4):
                    out_ref[q * QB:(q + 1) * QB,
                            c * D + s * SD:(c * D + (s + 1) * SD)] = (
                        g_ref[c, s, q].astype(jnp.float32)
                        * inv[q * QB:(q + 1) * QB])

    return pl.pallas_call(
        body,
        out_shape=jax.ShapeDtypeStruct((T, N_FULL), jnp.float32),
        in_specs=[
            pl.BlockSpec(memory_space=pltpu.VMEM),
            pl.BlockSpec(memory_space=pltpu.VMEM),
        ],
        out_specs=pl.BlockSpec(memory_space=pltpu.VMEM),
        scratch_shapes=[pltpu.VMEM((NZ, S, 4, QB, SD), jnp.bfloat16)]
        + [pltpu.SemaphoreType.DMA((NH, S))] * 10,
        compiler_params=pltpu.CompilerParams(collective_id=0),
    )(x, W)
